# baseline (speedup 1.0000x reference)
import functools

import jax
import jax.numpy as jnp
from jax import lax
from jax.experimental import pallas as pl
from jax.experimental.pallas import tpu as pltpu
from jax.experimental.pallas import tpu_sc as plsc

_NC = 2
_NS = 16
_LANES = 16
_NB = 4
_NE = 8


def _make_agg(N: int, D: int, E: int):
    NW = _NC * _NS
    assert E % NW == 0, E
    e_per_w = E // NW
    K = 80
    assert e_per_w % K == 0 and K % _LANES == 0
    n_chunks = e_per_w // K
    G = D // _LANES
    n_zc = N // K
    assert N % K == 0
    assert n_chunks >= _NE >= 2 * _NB

    @functools.partial(
        pl.kernel,
        out_type=jax.ShapeDtypeStruct((_NC, N, D), jnp.float32),
        mesh=plsc.VectorSubcoreMesh(core_axis_name="c", subcore_axis_name="s"),
        scratch_types=[
            pltpu.VMEM((_NE, K), jnp.int32),
            pltpu.VMEM((_NE, K), jnp.int32),
            pltpu.VMEM((_NE, K), jnp.float32),
            pltpu.VMEM((_NB, K, D), jnp.float32),
            pltpu.VMEM_SHARED((N, D), jnp.float32),
        ] + [pltpu.SemaphoreType.DMA] * (2 * _NB + _NE + 1),
    )
    def agg(x_hbm, src_hbm, dst_hbm, ev_hbm, out_hbm,
            sidx_st, didx_st, evv_st, rowsb, acc, *sems):
        gsems = sems[:_NB]
        ssems = sems[_NB:2 * _NB]
        esems = sems[2 * _NB:2 * _NB + _NE]
        zsem = sems[2 * _NB + _NE]
        cid = lax.axis_index("c")
        sid = lax.axis_index("s")
        wid = sid * _NC + cid
        ebase = wid * e_per_w

        def e_start(j, eb):
            base = ebase + j * K
            pltpu.async_copy(src_hbm.at[pl.ds(base, K)], sidx_st.at[eb],
                             esems[eb])
            pltpu.async_copy(dst_hbm.at[pl.ds(base, K)], didx_st.at[eb],
                             esems[eb])
            pltpu.async_copy(ev_hbm.at[pl.ds(base, K)], evv_st.at[eb],
                             esems[eb])

        def e_wait(j, eb):
            base = ebase + j * K
            pltpu.make_async_copy(src_hbm.at[pl.ds(base, K)], sidx_st.at[eb],
                                  esems[eb]).wait()
            pltpu.make_async_copy(dst_hbm.at[pl.ds(base, K)], didx_st.at[eb],
                                  esems[eb]).wait()
            pltpu.make_async_copy(ev_hbm.at[pl.ds(base, K)], evv_st.at[eb],
                                  esems[eb]).wait()

        def g_start(idx, b, eb):
            pltpu.async_copy(x_hbm.at[sidx_st.at[eb]], rowsb.at[b], gsems[b])

        def g_wait(idx, b, eb):
            pltpu.make_async_copy(x_hbm.at[sidx_st.at[eb]], rowsb.at[b],
                                  gsems[b]).wait()

        def s_start(idx, b, eb):
            pltpu.async_copy(rowsb.at[b], acc.at[didx_st.at[eb]], ssems[b],
                             add=True)

        def s_wait(idx, b, eb):
            pltpu.make_async_copy(rowsb.at[b], acc.at[didx_st.at[eb]],
                                  ssems[b]).wait()

        for j in range(_NE):
            e_start(j, j)

        def _zrow(j, _):
            for g in range(G):
                rowsb[0, j, pl.ds(g * _LANES, _LANES)] = jnp.zeros(
                    (_LANES,), jnp.float32)
            return 0
        lax.fori_loop(0, K, _zrow, 0)

        for t in range(-(-n_zc // _NS)):
            c = sid + t * _NS
            @pl.when(c < n_zc)
            def _():
                pltpu.async_copy(rowsb.at[0], acc.at[pl.ds(c * K, K)], zsem)
        for t in range(-(-n_zc // _NS)):
            c = sid + t * _NS
            @pl.when(c < n_zc)
            def _():
                pltpu.make_async_copy(rowsb.at[0], acc.at[pl.ds(c * K, K)],
                                      zsem).wait()

        for p in range(_NB - 1):
            e_wait(p, p)
            g_start(p, p, p)
        plsc.subcore_barrier()

        def _oct(t, _):
            for u in range(_NE):
                idx = _NE * t + u
                b = u % _NB
                eb = u
                @pl.when(idx < n_chunks)
                def _():
                    g_wait(idx, b, eb)
                    nb = (u + _NB - 1) % _NB
                    neb = (u + _NE - 1) % _NE
                    web = (u + _NB - 1) % _NE
                    @pl.when(idx >= 1)
                    def _():
                        s_wait(idx - 1, nb, neb)
                        @pl.when(idx - 1 + _NE < n_chunks)
                        def _():
                            e_start(idx - 1 + _NE, neb)
                    @pl.when(idx + _NB - 1 < n_chunks)
                    def _():
                        e_wait(idx + _NB - 1, web)
                        g_start(idx + _NB - 1, nb, web)

                    for jj in range(K // _LANES):
                        ev16 = evv_st[eb, pl.ds(jj * _LANES, _LANES)]
                        for j in range(_LANES):
                            e = ev16[j]
                            r = jj * _LANES + j
                            for g in range(G):
                                sl = pl.ds(g * _LANES, _LANES)
                                rowsb[b, r, sl] = rowsb[b, r, sl] * e

                    s_start(idx, b, eb)
            return 0
        lax.fori_loop(0, -(-n_chunks // _NE), _oct, 0)
        s_wait(n_chunks - 1, (n_chunks - 1) % _NB, (n_chunks - 1) % _NE)
        plsc.subcore_barrier()

        for t in range(-(-n_zc // _NS)):
            c = sid + t * _NS
            @pl.when(c < n_zc)
            def _():
                pltpu.async_copy(acc.at[pl.ds(c * K, K)],
                                 out_hbm.at[cid, pl.ds(c * K, K)], zsem)
        for t in range(-(-n_zc // _NS)):
            c = sid + t * _NS
            @pl.when(c < n_zc)
            def _():
                pltpu.make_async_copy(acc.at[pl.ds(c * K, K)],
                                      out_hbm.at[cid, pl.ds(c * K, K)],
                                      zsem).wait()

    return agg, e_per_w, n_chunks, K


def _tc_finish(p_ref, w_ref, o_ref):
    s = p_ref[0] + p_ref[1]
    y = lax.dot_general(s, w_ref[...], (((1,), (1,)), ((), ())),
                        preferred_element_type=jnp.float32,
                        precision=lax.Precision.HIGHEST)
    o_ref[...] = jnp.maximum(y, 0.0)


def kernel(x, edge_index, edge_values, W):
    N, D = x.shape
    E = edge_values.shape[0]
    dst = edge_index[0].astype(jnp.int32)
    src = edge_index[1].astype(jnp.int32)

    agg, e_per_w, n_chunks, K = _make_agg(N, D, E)
    partials = agg(x, src, dst, edge_values)

    BR = 1000
    assert N % BR == 0
    out = pl.pallas_call(
        _tc_finish,
        grid=(N // BR,),
        in_specs=[
            pl.BlockSpec((_NC, BR, D), lambda i: (0, i, 0)),
            pl.BlockSpec((D, D), lambda i: (0, 0)),
        ],
        out_specs=pl.BlockSpec((BR, D), lambda i: (i, 0)),
        out_shape=jax.ShapeDtypeStruct((N, D), jnp.float32),
    )(partials, W)
    return out


# speedup vs baseline: 1.5817x; 1.5817x over previous
import functools

import jax
import jax.numpy as jnp
from jax import lax
from jax.experimental import pallas as pl
from jax.experimental.pallas import tpu as pltpu
from jax.experimental.pallas import tpu_sc as plsc

_NC = 2
_NS = 16
_LANES = 16
_NB = 4
_NE = 8


def _make_agg(N: int, D: int, E: int):
    NW = _NC * _NS
    assert E % NW == 0, E
    e_per_w = E // NW
    K = 80
    assert e_per_w % K == 0 and K % _LANES == 0
    n_chunks = e_per_w // K
    G = D // _LANES
    n_zc = N // K
    assert N % K == 0
    assert n_chunks >= _NE >= 2 * _NB

    @functools.partial(
        pl.kernel,
        out_type=jax.ShapeDtypeStruct((_NC, N, D), jnp.float32),
        mesh=plsc.VectorSubcoreMesh(core_axis_name="c", subcore_axis_name="s"),
        scratch_types=[
            pltpu.VMEM((_NE, K), jnp.int32),
            pltpu.VMEM((_NE, K), jnp.int32),
            pltpu.VMEM((_NE, K), jnp.float32),
            pltpu.VMEM((_NB, K, D), jnp.float32),
            pltpu.VMEM_SHARED((N, D), jnp.float32),
        ] + [pltpu.SemaphoreType.DMA] * (2 * _NB + _NE + 1),
    )
    def agg(x_hbm, src_hbm, dst_hbm, ev_hbm, out_hbm,
            sidx_st, didx_st, evv_st, rowsb, acc, *sems):
        gsems = sems[:_NB]
        ssems = sems[_NB:2 * _NB]
        esems = sems[2 * _NB:2 * _NB + _NE]
        zsem = sems[2 * _NB + _NE]
        cid = lax.axis_index("c")
        sid = lax.axis_index("s")
        wid = sid * _NC + cid
        ebase = wid * e_per_w

        def e_start(j, eb):
            base = ebase + j * K
            pltpu.async_copy(src_hbm.at[pl.ds(base, K)], sidx_st.at[eb],
                             esems[eb])
            pltpu.async_copy(dst_hbm.at[pl.ds(base, K)], didx_st.at[eb],
                             esems[eb])
            pltpu.async_copy(ev_hbm.at[pl.ds(base, K)], evv_st.at[eb],
                             esems[eb])

        def e_wait(j, eb):
            base = ebase + j * K
            pltpu.make_async_copy(src_hbm.at[pl.ds(base, K)], sidx_st.at[eb],
                                  esems[eb]).wait()
            pltpu.make_async_copy(dst_hbm.at[pl.ds(base, K)], didx_st.at[eb],
                                  esems[eb]).wait()
            pltpu.make_async_copy(ev_hbm.at[pl.ds(base, K)], evv_st.at[eb],
                                  esems[eb]).wait()

        def g_start(idx, b, eb):
            pltpu.async_copy(x_hbm.at[sidx_st.at[eb]], rowsb.at[b], gsems[b])

        def g_wait(idx, b, eb):
            pltpu.make_async_copy(x_hbm.at[sidx_st.at[eb]], rowsb.at[b],
                                  gsems[b]).wait()

        def s_start(idx, b, eb):
            pltpu.async_copy(rowsb.at[b], acc.at[didx_st.at[eb]], ssems[b],
                             add=True)

        def s_wait(idx, b, eb):
            pltpu.make_async_copy(rowsb.at[b], acc.at[didx_st.at[eb]],
                                  ssems[b]).wait()

        for j in range(_NE):
            e_start(j, j)

        def _zrow(j, _):
            for g in range(G):
                rowsb[0, j, pl.ds(g * _LANES, _LANES)] = jnp.zeros(
                    (_LANES,), jnp.float32)
            return 0
        lax.fori_loop(0, K, _zrow, 0)

        for t in range(-(-n_zc // _NS)):
            c = sid + t * _NS
            @pl.when(c < n_zc)
            def _():
                pltpu.async_copy(rowsb.at[0], acc.at[pl.ds(c * K, K)], zsem)
        for t in range(-(-n_zc // _NS)):
            c = sid + t * _NS
            @pl.when(c < n_zc)
            def _():
                pltpu.make_async_copy(rowsb.at[0], acc.at[pl.ds(c * K, K)],
                                      zsem).wait()

        for p in range(_NB - 2):
            e_wait(p, p)
            g_start(p, p, p)
        plsc.subcore_barrier()

        def _oct(t, _):
            for u in range(_NE):
                idx = _NE * t + u
                b = u % _NB
                eb = u
                @pl.when(idx < n_chunks)
                def _():
                    g_wait(idx, b, eb)
                    nb = (u + _NB - 2) % _NB
                    neb = (u + _NE - 2) % _NE
                    web = (u + _NB - 2) % _NE
                    @pl.when(idx >= 2)
                    def _():
                        s_wait(idx - 2, nb, neb)
                        @pl.when(idx - 2 + _NE < n_chunks)
                        def _():
                            e_start(idx - 2 + _NE, neb)
                    @pl.when(idx + _NB - 2 < n_chunks)
                    def _():
                        e_wait(idx + _NB - 2, web)
                        g_start(idx + _NB - 2, nb, web)

                    def _scale(jj, _):
                        ev16 = evv_st[eb, pl.ds(jj * _LANES, _LANES)]
                        for j in range(_LANES):
                            e = ev16[j]
                            r = jj * _LANES + j
                            for g in range(G):
                                sl = pl.ds(g * _LANES, _LANES)
                                rowsb[b, r, sl] = rowsb[b, r, sl] * e
                        return 0
                    lax.fori_loop(0, K // _LANES, _scale, 0)

                    s_start(idx, b, eb)
            return 0
        lax.fori_loop(0, -(-n_chunks // _NE), _oct, 0)
        s_wait(n_chunks - 2, (n_chunks - 2) % _NB, (n_chunks - 2) % _NE)
        s_wait(n_chunks - 1, (n_chunks - 1) % _NB, (n_chunks - 1) % _NE)
        plsc.subcore_barrier()

        for t in range(-(-n_zc // _NS)):
            c = sid + t * _NS
            @pl.when(c < n_zc)
            def _():
                pltpu.async_copy(acc.at[pl.ds(c * K, K)],
                                 out_hbm.at[cid, pl.ds(c * K, K)], zsem)
        for t in range(-(-n_zc // _NS)):
            c = sid + t * _NS
            @pl.when(c < n_zc)
            def _():
                pltpu.make_async_copy(acc.at[pl.ds(c * K, K)],
                                      out_hbm.at[cid, pl.ds(c * K, K)],
                                      zsem).wait()

    return agg, e_per_w, n_chunks, K


def _tc_finish(p_ref, w_ref, o_ref):
    s = p_ref[0] + p_ref[1]
    y = lax.dot_general(s, w_ref[...], (((1,), (1,)), ((), ())),
                        preferred_element_type=jnp.float32,
                        precision=lax.Precision.HIGHEST)
    o_ref[...] = jnp.maximum(y, 0.0)


def kernel(x, edge_index, edge_values, W):
    N, D = x.shape
    E = edge_values.shape[0]
    dst = edge_index[0].astype(jnp.int32)
    src = edge_index[1].astype(jnp.int32)

    agg, e_per_w, n_chunks, K = _make_agg(N, D, E)
    partials = agg(x, src, dst, edge_values)

    BR = 1000
    assert N % BR == 0
    out = pl.pallas_call(
        _tc_finish,
        grid=(N // BR,),
        in_specs=[
            pl.BlockSpec((_NC, BR, D), lambda i: (0, i, 0)),
            pl.BlockSpec((D, D), lambda i: (0, 0)),
        ],
        out_specs=pl.BlockSpec((BR, D), lambda i: (i, 0)),
        out_shape=jax.ShapeDtypeStruct((N, D), jnp.float32),
    )(partials, W)
    return out
